# Initial kernel scaffold; baseline (speedup 1.0000x reference)
import functools

import jax
import jax.numpy as jnp
from jax import lax
from jax.experimental import pallas as pl
from jax.experimental.pallas import tpu as pltpu
from jax.experimental.pallas import tpu_sc as plsc

B = 4096
L = 200
D = 64
NC = 2
NS = 16
NW = NC * NS
ROWS_PER_W = B // NW
CHUNK = 16
NCHUNK = ROWS_PER_W // CHUNK
UNROLL = 8


def _body(x_hbm, table_hbm, out_hbm, idx_v, rows_v, t0_v, out_v, sem0, sem1):
    wid = lax.axis_index("s") * NC + lax.axis_index("c")
    base = wid * ROWS_PER_W

    pltpu.sync_copy(table_hbm.at[pl.ds(0, 1)], t0_v)
    t0 = [t0_v[0, pl.ds(k * 16, 16)] for k in range(4)]

    sems = (sem0, sem1)

    def start_gather(r, slot):
        pltpu.async_copy(table_hbm.at[idx_v.at[r]], rows_v.at[slot], sems[slot])

    def wait_gather(r, slot):
        pltpu.make_async_copy(
            table_hbm.at[idx_v.at[r]], rows_v.at[slot], sems[slot]
        ).wait()

    lane = lax.iota(jnp.int32, 16)

    def chunk_body(c, carry):
        cbase = base + c * CHUNK
        pltpu.sync_copy(x_hbm.at[pl.ds(cbase, CHUNK)], idx_v)
        start_gather(0, 0)
        for r in range(CHUNK):
            slot = r % 2
            if r + 1 < CHUNK:
                start_gather(r + 1, (r + 1) % 2)

            zacc = jnp.zeros((16,), jnp.float32)
            for kk in range(12):
                v = idx_v[r, pl.ds(kk * 16, 16)]
                zacc = zacc + jnp.where(v == 0, 1.0, 0.0)
            vtail = idx_v[r, pl.ds(L - 16, 16)]
            zacc = zacc + jnp.where((vtail == 0) & (lane >= 8), 1.0, 0.0)
            npad = jnp.sum(zacc)
            npad_v = jnp.full((16,), npad, jnp.float32)
            recip_v = 1.0 / jnp.maximum(float(L) - npad_v, 1.0)

            wait_gather(r, slot)

            def acc_body(i, acc):
                a0, a1, a2, a3 = acc
                for jj in range(UNROLL):
                    j = i * UNROLL + jj
                    a0 = a0 + rows_v[slot, j, pl.ds(0, 16)]
                    a1 = a1 + rows_v[slot, j, pl.ds(16, 16)]
                    a2 = a2 + rows_v[slot, j, pl.ds(32, 16)]
                    a3 = a3 + rows_v[slot, j, pl.ds(48, 16)]
                return (a0, a1, a2, a3)

            zero = jnp.zeros((16,), jnp.float32)
            accs = lax.fori_loop(0, L // UNROLL, acc_body, (zero, zero, zero, zero))
            for k in range(4):
                out_v[r, pl.ds(k * 16, 16)] = (accs[k] - npad_v * t0[k]) * recip_v
        pltpu.sync_copy(out_v, out_hbm.at[pl.ds(cbase, CHUNK)])
        return carry

    lax.fori_loop(0, NCHUNK, chunk_body, 0)


@functools.partial(
    pl.kernel,
    out_type=jax.ShapeDtypeStruct((B, D), jnp.float32),
    mesh=plsc.VectorSubcoreMesh(core_axis_name="c", subcore_axis_name="s"),
    scratch_types=[
        pltpu.VMEM((CHUNK, L), jnp.int32),
        pltpu.VMEM((2, L, D), jnp.float32),
        pltpu.VMEM((1, D), jnp.float32),
        pltpu.VMEM((CHUNK, D), jnp.float32),
        pltpu.SemaphoreType.DMA,
        pltpu.SemaphoreType.DMA,
    ],
)
def _encoder_kernel(x_hbm, table_hbm, out_hbm, idx_v, rows_v, t0_v, out_v, s0, s1):
    _body(x_hbm, table_hbm, out_hbm, idx_v, rows_v, t0_v, out_v, s0, s1)


def kernel(x, table):
    return _encoder_kernel(x.astype(jnp.int32), table)


# speedup vs baseline: 1.0490x; 1.0490x over previous
import functools

import jax
import jax.numpy as jnp
from jax import lax
from jax.experimental import pallas as pl
from jax.experimental.pallas import tpu as pltpu
from jax.experimental.pallas import tpu_sc as plsc

B = 4096
L = 200
D = 64
NC = 2
NS = 16
NW = NC * NS
ROWS_PER_W = B // NW
CHUNK = 16
NCHUNK = ROWS_PER_W // CHUNK
UNROLL = 8


def _body(x_hbm, table_hbm, out_hbm, idx_v, rows_v, t0_v, out_v, sem0, sem1):
    wid = lax.axis_index("s") * NC + lax.axis_index("c")
    base = wid * ROWS_PER_W

    pltpu.sync_copy(table_hbm.at[pl.ds(0, 1)], t0_v)
    t0 = [t0_v[0, pl.ds(k * 16, 16)] for k in range(4)]

    sems = (sem0, sem1)

    def start_gather(r, slot):
        pltpu.async_copy(table_hbm.at[idx_v.at[r]], rows_v.at[slot], sems[slot])

    def wait_gather(r, slot):
        pltpu.make_async_copy(
            table_hbm.at[idx_v.at[r]], rows_v.at[slot], sems[slot]
        ).wait()

    lane = lax.iota(jnp.int32, 16)

    def chunk_body(c, carry):
        cbase = base + c * CHUNK
        pltpu.sync_copy(x_hbm.at[pl.ds(cbase, CHUNK)], idx_v)
        start_gather(0, 0)
        for r in range(CHUNK):
            slot = r % 2
            if r + 1 < CHUNK:
                start_gather(r + 1, (r + 1) % 2)

            npad_i = jnp.zeros((16,), jnp.int32)
            for kk in range(12):
                v = idx_v[r, pl.ds(kk * 16, 16)]
                npad_i = npad_i + plsc.all_reduce_population_count(v == 0)
            vtail = idx_v[r, pl.ds(L - 16, 16)]
            npad_i = npad_i + plsc.all_reduce_population_count(
                (vtail == 0) & (lane >= 8)
            )
            npad_v = npad_i.astype(jnp.float32)
            recip_v = 1.0 / jnp.maximum(float(L) - npad_v, 1.0)

            wait_gather(r, slot)

            def acc_body(i, acc):
                a0, a1, a2, a3 = acc
                for jj in range(UNROLL):
                    j = i * UNROLL + jj
                    a0 = a0 + rows_v[slot, j, pl.ds(0, 16)]
                    a1 = a1 + rows_v[slot, j, pl.ds(16, 16)]
                    a2 = a2 + rows_v[slot, j, pl.ds(32, 16)]
                    a3 = a3 + rows_v[slot, j, pl.ds(48, 16)]
                return (a0, a1, a2, a3)

            zero = jnp.zeros((16,), jnp.float32)
            accs = lax.fori_loop(0, L // UNROLL, acc_body, (zero, zero, zero, zero))
            for k in range(4):
                out_v[r, pl.ds(k * 16, 16)] = (accs[k] - npad_v * t0[k]) * recip_v
        pltpu.sync_copy(out_v, out_hbm.at[pl.ds(cbase, CHUNK)])
        return carry

    lax.fori_loop(0, NCHUNK, chunk_body, 0)


@functools.partial(
    pl.kernel,
    out_type=jax.ShapeDtypeStruct((B, D), jnp.float32),
    mesh=plsc.VectorSubcoreMesh(core_axis_name="c", subcore_axis_name="s"),
    compiler_params=pltpu.CompilerParams(
        needs_layout_passes=False, use_tc_tiling_on_sc=False
    ),
    scratch_types=[
        pltpu.VMEM((CHUNK, L), jnp.int32),
        pltpu.VMEM((2, L, D), jnp.float32),
        pltpu.VMEM((1, D), jnp.float32),
        pltpu.VMEM((CHUNK, D), jnp.float32),
        pltpu.SemaphoreType.DMA,
        pltpu.SemaphoreType.DMA,
    ],
)
def _encoder_kernel(x_hbm, table_hbm, out_hbm, idx_v, rows_v, t0_v, out_v, s0, s1):
    _body(x_hbm, table_hbm, out_hbm, idx_v, rows_v, t0_v, out_v, s0, s1)


def kernel(x, table):
    return _encoder_kernel(x.astype(jnp.int32), table)
